# baseline (speedup 1.0000x reference)
import jax
import jax.numpy as jnp
from jax.experimental import pallas as pl

DIM = 256
NUM_Q = 2
CODEBOOK_SIZE = 256
COMMIT_W = 0.25
IN_DIM = 784
BW = 8


def _mm_t(w, a):
    return jax.lax.dot_general(w, a, (((0,), (0,)), ((), ())),
                               preferred_element_type=jnp.float32)


def _fused_body(x_ref, ew1_ref, eb1_ref, ew2_ref, eb2_ref,
                cb_ref, cbh_ref, cbm_ref,
                dw1_ref, db1_ref, dw2_ref, db2_ref,
                recon_ref, idx_ref, loss_ref, *, batch_total):
    i = pl.program_id(0)
    cols = BW * 128
    xb = x_ref[...].reshape(IN_DIM, cols)

    h = jnp.maximum(_mm_t(ew1_ref[...], xb) + eb1_ref[...], 0.0)
    e = _mm_t(ew2_ref[...], h) + eb2_ref[...]

    siota = jax.lax.broadcasted_iota(jnp.int32, (CODEBOOK_SIZE, cols), 0)
    resid = e
    qsum = jnp.zeros_like(e)
    commit_sum = jnp.float32(0.0)
    idx_list = []
    for q in range(NUM_Q):
        cbq = cb_ref[q]
        cn = jnp.sum(cbq * cbq, axis=1, keepdims=True)
        rn = jnp.sum(resid * resid, axis=0, keepdims=True)
        rc = jnp.dot(cbq, resid, preferred_element_type=jnp.float32)
        d = (rn - 2.0 * rc) + cn
        dmin = jnp.min(d, axis=0, keepdims=True)
        idx = jnp.min(jnp.where(d == dmin, siota, CODEBOOK_SIZE),
                      axis=0, keepdims=True)
        if q == 0:
            onehot = (siota == idx).astype(jnp.float32)
            quant = jax.lax.dot_general(
                cbq, onehot, (((0,), (0,)), ((), ())),
                precision=jax.lax.Precision.HIGHEST)
        else:
            onehot = (siota == idx).astype(jnp.bfloat16)
            quant = (_mm_t(cbh_ref[q], onehot)
                     + _mm_t(cbm_ref[q], onehot))
        diff = resid - quant
        commit_sum = commit_sum + jnp.sum(diff * diff)
        qsum = qsum + quant
        resid = diff
        idx_list.append(idx)

    dh = jnp.maximum(_mm_t(dw1_ref[...], qsum) + db1_ref[...], 0.0)
    rec = jnp.tanh(_mm_t(dw2_ref[...], dh) + db2_ref[...])

    recon_ref[...] = rec.reshape(IN_DIM, BW, 128)
    idx_ref[...] = jnp.concatenate(idx_list, axis=0)

    dx = rec - xb
    part = (jnp.sum(dx * dx) / (batch_total * float(IN_DIM))
            + COMMIT_W * commit_sum / (batch_total * float(DIM)))

    @pl.when(i == 0)
    def _init():
        loss_ref[...] = jnp.zeros_like(loss_ref)

    loss_ref[...] = loss_ref[...] + part


def kernel(x, enc_w1, enc_b1, enc_w2, enc_b2, codebooks,
           dec_w1, dec_b1, dec_w2, dec_b2, *, interpret=False):
    b = x.shape[0]
    bt = b // 128
    xt3 = jnp.transpose(x, (2, 3, 1, 0)).reshape(IN_DIM, bt, 128)
    grid = (bt // BW,)

    cbh = codebooks.astype(jnp.bfloat16)
    cbm = (codebooks - cbh.astype(jnp.float32)).astype(jnp.bfloat16)

    full = lambda a: pl.BlockSpec(a.shape, lambda i: (0,) * a.ndim)
    col_bias = lambda n: pl.BlockSpec((n, 1), lambda i: (0, 0))
    rect3, idxt, loss = pl.pallas_call(
        lambda *refs: _fused_body(*refs, batch_total=float(b)),
        grid=grid,
        in_specs=[
            pl.BlockSpec((IN_DIM, BW, 128), lambda i: (0, i, 0)),
            full(enc_w1), col_bias(512),
            full(enc_w2), col_bias(256),
            full(codebooks), full(cbh), full(cbm),
            full(dec_w1), col_bias(512),
            full(dec_w2), col_bias(IN_DIM),
        ],
        out_specs=[
            pl.BlockSpec((IN_DIM, BW, 128), lambda i: (0, i, 0)),
            pl.BlockSpec((NUM_Q, BW * 128), lambda i: (0, i)),
            pl.BlockSpec((1, 1), lambda i: (0, 0)),
        ],
        out_shape=[
            jax.ShapeDtypeStruct((IN_DIM, bt, 128), jnp.float32),
            jax.ShapeDtypeStruct((NUM_Q, b), jnp.int32),
            jax.ShapeDtypeStruct((1, 1), jnp.float32),
        ],
        interpret=interpret,
    )(xt3, enc_w1, enc_b1.reshape(-1, 1),
      enc_w2, enc_b2.reshape(-1, 1),
      codebooks, cbh, cbm,
      dec_w1, dec_b1.reshape(-1, 1),
      dec_w2, dec_b2.reshape(-1, 1))

    recon = jnp.transpose(rect3.reshape(IN_DIM, 1, bt, 128),
                          (2, 3, 1, 0)).reshape(b, 1, 28, 28)
    return (recon, idxt.T, loss[0, 0])


# speedup vs baseline: 1.0941x; 1.0941x over previous
import jax
import jax.numpy as jnp
from jax.experimental import pallas as pl

DIM = 256
NUM_Q = 2
CODEBOOK_SIZE = 256
COMMIT_W = 0.25
IN_DIM = 784
BW = 16


def _mm_t(w, a):
    return jax.lax.dot_general(w, a, (((0,), (0,)), ((), ())),
                               preferred_element_type=jnp.float32)


def _fused_body(x_ref, ew1_ref, eb1_ref, ew2_ref, eb2_ref,
                cb_ref, cbh_ref, cbm_ref,
                dw1_ref, db1_ref, dw2_ref, db2_ref,
                recon_ref, idx_ref, loss_ref, *, batch_total):
    i = pl.program_id(0)
    cols = BW * 128
    xb = x_ref[...].reshape(IN_DIM, cols)

    h = jnp.maximum(_mm_t(ew1_ref[...], xb) + eb1_ref[...], 0.0)
    e = _mm_t(ew2_ref[...], h) + eb2_ref[...]

    siota = jax.lax.broadcasted_iota(jnp.int32, (CODEBOOK_SIZE, cols), 0)
    resid = e
    qsum = jnp.zeros_like(e)
    commit_sum = jnp.float32(0.0)
    idx_list = []
    for q in range(NUM_Q):
        cbq = cb_ref[q]
        cn = jnp.sum(cbq * cbq, axis=1, keepdims=True)
        rn = jnp.sum(resid * resid, axis=0, keepdims=True)
        if q > 0:
            commit_sum = commit_sum + jnp.sum(rn)
        rc = jnp.dot(cbq, resid, preferred_element_type=jnp.float32)
        d = (rn - 2.0 * rc) + cn
        dmin = jnp.min(d, axis=0, keepdims=True)
        idx = jnp.min(jnp.where(d == dmin, siota, CODEBOOK_SIZE),
                      axis=0, keepdims=True)
        if q == 0:
            onehot = (siota == idx).astype(jnp.float32)
            quant = jax.lax.dot_general(
                cbq, onehot, (((0,), (0,)), ((), ())),
                precision=jax.lax.Precision.HIGHEST)
        else:
            onehot = (siota == idx).astype(jnp.bfloat16)
            quant = (_mm_t(cbh_ref[q], onehot)
                     + _mm_t(cbm_ref[q], onehot))
        diff = resid - quant
        qsum = qsum + quant
        resid = diff
        idx_list.append(idx)
    commit_sum = commit_sum + jnp.sum(resid * resid)

    dh = jnp.maximum(_mm_t(dw1_ref[...], qsum) + db1_ref[...], 0.0)
    rec = jnp.tanh(_mm_t(dw2_ref[...], dh) + db2_ref[...])

    recon_ref[...] = rec.reshape(IN_DIM, BW, 128)
    idx_ref[...] = jnp.concatenate(idx_list, axis=0)

    dx = rec - xb
    part = (jnp.sum(dx * dx) / (batch_total * float(IN_DIM))
            + COMMIT_W * commit_sum / (batch_total * float(DIM)))

    @pl.when(i == 0)
    def _init():
        loss_ref[...] = jnp.zeros_like(loss_ref)

    loss_ref[...] = loss_ref[...] + part


def kernel(x, enc_w1, enc_b1, enc_w2, enc_b2, codebooks,
           dec_w1, dec_b1, dec_w2, dec_b2, *, interpret=False):
    b = x.shape[0]
    bt = b // 128
    xt3 = jnp.transpose(x, (2, 3, 1, 0)).reshape(IN_DIM, bt, 128)
    grid = (bt // BW,)

    cbh = codebooks.astype(jnp.bfloat16)
    cbm = (codebooks - cbh.astype(jnp.float32)).astype(jnp.bfloat16)

    full = lambda a: pl.BlockSpec(a.shape, lambda i: (0,) * a.ndim)
    col_bias = lambda n: pl.BlockSpec((n, 1), lambda i: (0, 0))
    rect3, idxt, loss = pl.pallas_call(
        lambda *refs: _fused_body(*refs, batch_total=float(b)),
        grid=grid,
        in_specs=[
            pl.BlockSpec((IN_DIM, BW, 128), lambda i: (0, i, 0)),
            full(enc_w1), col_bias(512),
            full(enc_w2), col_bias(256),
            full(codebooks), full(cbh), full(cbm),
            full(dec_w1), col_bias(512),
            full(dec_w2), col_bias(IN_DIM),
        ],
        out_specs=[
            pl.BlockSpec((IN_DIM, BW, 128), lambda i: (0, i, 0)),
            pl.BlockSpec((NUM_Q, BW * 128), lambda i: (0, i)),
            pl.BlockSpec((1, 1), lambda i: (0, 0)),
        ],
        out_shape=[
            jax.ShapeDtypeStruct((IN_DIM, bt, 128), jnp.float32),
            jax.ShapeDtypeStruct((NUM_Q, b), jnp.int32),
            jax.ShapeDtypeStruct((1, 1), jnp.float32),
        ],
        interpret=interpret,
    )(xt3, enc_w1, enc_b1.reshape(-1, 1),
      enc_w2, enc_b2.reshape(-1, 1),
      codebooks, cbh, cbm,
      dec_w1, dec_b1.reshape(-1, 1),
      dec_w2, dec_b2.reshape(-1, 1))

    recon = jnp.transpose(rect3.reshape(IN_DIM, 1, bt, 128),
                          (2, 3, 1, 0)).reshape(b, 1, 28, 28)
    return (recon, idxt.T, loss[0, 0])
